# Initial kernel scaffold; baseline (speedup 1.0000x reference)
import jax
import jax.numpy as jnp
from jax.experimental import pallas as pl


def kernel(x):
    raise NotImplementedError("write your pallas kernel here")


# speedup vs baseline: 1.1056x; 1.1056x over previous
import functools

import jax
import jax.numpy as jnp
from jax import lax
from jax.experimental import pallas as pl
from jax.experimental.pallas import tpu as pltpu
from jax.experimental.pallas import tpu_sc as plsc

ROWS = 128
COLS = 32768
NUM_CORES = 2
NUM_SUBCORES = 16
NUM_WORKERS = NUM_CORES * NUM_SUBCORES
ROWS_PER_WORKER = ROWS // NUM_WORKERS
LANES = 16
NVECS = COLS // LANES


def _sc_row_cumsum(x_flat):
    mesh = plsc.VectorSubcoreMesh(
        core_axis_name="c", subcore_axis_name="s")

    @functools.partial(
        pl.kernel,
        out_type=jax.ShapeDtypeStruct((ROWS * COLS,), jnp.float32),
        mesh=mesh,
        scratch_types=[pltpu.VMEM((COLS,), jnp.float32)],
        compiler_params=pltpu.CompilerParams(needs_layout_passes=False),
    )
    def k(x_hbm, out_hbm, buf):
        wid = lax.axis_index("s") * NUM_CORES + lax.axis_index("c")
        idx_last = jnp.full((LANES,), LANES - 1, jnp.int32)

        for r in range(ROWS_PER_WORKER):
            base = (wid * ROWS_PER_WORKER + r) * COLS
            pltpu.sync_copy(x_hbm.at[pl.ds(base, COLS)], buf)

            def body(i, carry):
                sl = pl.ds(i * LANES, LANES)
                s = plsc.cumsum(buf[sl])
                o = s + carry
                buf[sl] = o
                return o.at[idx_last].get(mode="promise_in_bounds")

            lax.fori_loop(0, NVECS, body, jnp.zeros((LANES,), jnp.float32))
            pltpu.sync_copy(buf, out_hbm.at[pl.ds(base, COLS)])

    return k(x_flat)


def kernel(x):
    return _sc_row_cumsum(x.reshape(-1)).reshape(ROWS, COLS)


# speedup vs baseline: 2.1645x; 1.9578x over previous
import functools

import jax
import jax.numpy as jnp
from jax import lax
from jax.experimental import pallas as pl
from jax.experimental.pallas import tpu as pltpu
from jax.experimental.pallas import tpu_sc as plsc

ROWS = 128
COLS = 32768
NUM_CORES = 2
NUM_SUBCORES = 16
NUM_WORKERS = NUM_CORES * NUM_SUBCORES
ROWS_PER_WORKER = ROWS // NUM_WORKERS
LANES = 16
NVECS = COLS // LANES
UNROLL = 8
NGROUPS = NVECS // UNROLL


def _sc_row_cumsum(x_flat):
    mesh = plsc.VectorSubcoreMesh(
        core_axis_name="c", subcore_axis_name="s")

    @functools.partial(
        pl.kernel,
        out_type=jax.ShapeDtypeStruct((ROWS * COLS,), jnp.float32),
        mesh=mesh,
        scratch_types=[
            pltpu.VMEM((2, COLS), jnp.float32),
            pltpu.SemaphoreType.DMA,
            pltpu.SemaphoreType.DMA,
            pltpu.SemaphoreType.DMA,
            pltpu.SemaphoreType.DMA,
        ],
        compiler_params=pltpu.CompilerParams(needs_layout_passes=False),
    )
    def k(x_hbm, out_hbm, buf, in_sem0, in_sem1, out_sem0, out_sem1):
        wid = lax.axis_index("s") * NUM_CORES + lax.axis_index("c")
        idx_last = jnp.full((LANES,), LANES - 1, jnp.int32)
        in_sems = (in_sem0, in_sem1)
        out_sems = (out_sem0, out_sem1)

        def row_base(r):
            return (wid * ROWS_PER_WORKER + r) * COLS

        def scan_row(b):
            def group_body(g, c):
                base = g * (UNROLL * LANES)
                sls = [pl.ds(base + j * LANES, LANES) for j in range(UNROLL)]
                ss = [plsc.cumsum(buf[b, sl]) for sl in sls]
                ts = [s.at[idx_last].get(mode="promise_in_bounds")
                      for s in ss]
                t01 = ts[0] + ts[1]
                t23 = ts[2] + ts[3]
                t45 = ts[4] + ts[5]
                t67 = ts[6] + ts[7]
                e4 = t01 + t23
                e = [None, ts[0], t01, t01 + ts[2],
                     e4, e4 + ts[4], e4 + t45, e4 + t45 + ts[6]]
                total = e4 + (t45 + t67)
                pres = [c] + [c + e[j] for j in range(1, UNROLL)]
                for j in range(UNROLL):
                    buf[b, sls[j]] = ss[j] + pres[j]
                return c + total

            lax.fori_loop(0, NGROUPS, group_body,
                          jnp.zeros((LANES,), jnp.float32))

        pending_out = [None, None]
        copy_in = pltpu.async_copy(
            x_hbm.at[pl.ds(row_base(0), COLS)], buf.at[0], in_sems[0])
        for r in range(ROWS_PER_WORKER):
            b = r % 2
            nb = (r + 1) % 2
            if r + 1 < ROWS_PER_WORKER:
                if pending_out[nb] is not None:
                    pending_out[nb].wait()
                    pending_out[nb] = None
                next_in = pltpu.async_copy(
                    x_hbm.at[pl.ds(row_base(r + 1), COLS)],
                    buf.at[nb], in_sems[nb])
            copy_in.wait()
            scan_row(b)
            pending_out[b] = pltpu.async_copy(
                buf.at[b], out_hbm.at[pl.ds(row_base(r), COLS)], out_sems[b])
            if r + 1 < ROWS_PER_WORKER:
                copy_in = next_in
        for p in pending_out:
            if p is not None:
                p.wait()

    return k(x_flat)


def kernel(x):
    return _sc_row_cumsum(x.reshape(-1)).reshape(ROWS, COLS)


# speedup vs baseline: 3.8513x; 1.7793x over previous
import functools

import jax
import jax.numpy as jnp
from jax import lax
from jax.experimental import pallas as pl
from jax.experimental.pallas import tpu as pltpu
from jax.experimental.pallas import tpu_sc as plsc

ROWS = 128
COLS = 32768
NUM_CORES = 2
NUM_SUBCORES = 16
NUM_WORKERS = NUM_CORES * NUM_SUBCORES
ROWS_PER_WORKER = ROWS // NUM_WORKERS
LANES = 16
NVECS = COLS // LANES
UNROLL = 8
NGROUPS = NVECS // UNROLL


def _sc_row_cumsum(x):
    mesh = plsc.VectorSubcoreMesh(
        core_axis_name="c", subcore_axis_name="s")

    @functools.partial(
        pl.kernel,
        out_type=jax.ShapeDtypeStruct((ROWS, COLS), jnp.float32),
        mesh=mesh,
        scratch_types=[
            pltpu.VMEM((2, COLS), jnp.float32),
            pltpu.SemaphoreType.DMA,
            pltpu.SemaphoreType.DMA,
            pltpu.SemaphoreType.DMA,
            pltpu.SemaphoreType.DMA,
        ],
        compiler_params=pltpu.CompilerParams(needs_layout_passes=False),
    )
    def k(x_hbm, out_hbm, buf, in_sem0, in_sem1, out_sem0, out_sem1):
        wid = lax.axis_index("s") * NUM_CORES + lax.axis_index("c")
        idx_last = jnp.full((LANES,), LANES - 1, jnp.int32)
        in_sems = (in_sem0, in_sem1)
        out_sems = (out_sem0, out_sem1)

        def row_idx(r):
            return wid * ROWS_PER_WORKER + r

        def scan_row(b):
            def group_body(g, c):
                base = g * (UNROLL * LANES)
                sls = [pl.ds(base + j * LANES, LANES) for j in range(UNROLL)]
                ss = [plsc.cumsum(buf[b, sl]) for sl in sls]
                ts = [s.at[idx_last].get(mode="promise_in_bounds")
                      for s in ss]
                t01 = ts[0] + ts[1]
                t23 = ts[2] + ts[3]
                t45 = ts[4] + ts[5]
                t67 = ts[6] + ts[7]
                e4 = t01 + t23
                e = [None, ts[0], t01, t01 + ts[2],
                     e4, e4 + ts[4], e4 + t45, e4 + t45 + ts[6]]
                total = e4 + (t45 + t67)
                pres = [c] + [c + e[j] for j in range(1, UNROLL)]
                for j in range(UNROLL):
                    buf[b, sls[j]] = ss[j] + pres[j]
                return c + total

            lax.fori_loop(0, NGROUPS, group_body,
                          jnp.zeros((LANES,), jnp.float32))

        pending_out = [None, None]
        copy_in = pltpu.async_copy(
            x_hbm.at[row_idx(0)], buf.at[0], in_sems[0])
        for r in range(ROWS_PER_WORKER):
            b = r % 2
            nb = (r + 1) % 2
            if r + 1 < ROWS_PER_WORKER:
                if pending_out[nb] is not None:
                    pending_out[nb].wait()
                    pending_out[nb] = None
                next_in = pltpu.async_copy(
                    x_hbm.at[row_idx(r + 1)], buf.at[nb], in_sems[nb])
            copy_in.wait()
            scan_row(b)
            pending_out[b] = pltpu.async_copy(
                buf.at[b], out_hbm.at[row_idx(r)], out_sems[b])
            if r + 1 < ROWS_PER_WORKER:
                copy_in = next_in
        for p in pending_out:
            if p is not None:
                p.wait()

    return k(x)


def kernel(x):
    return _sc_row_cumsum(x)


# speedup vs baseline: 3.9130x; 1.0160x over previous
import functools

import jax
import jax.numpy as jnp
from jax import lax
from jax.experimental import pallas as pl
from jax.experimental.pallas import tpu as pltpu
from jax.experimental.pallas import tpu_sc as plsc

ROWS = 128
COLS = 32768
NUM_CORES = 2
NUM_SUBCORES = 16
NUM_WORKERS = NUM_CORES * NUM_SUBCORES
ROWS_PER_WORKER = ROWS // NUM_WORKERS
LANES = 16
NVECS = COLS // LANES
UNROLL = 16
NGROUPS = NVECS // UNROLL


def _exclusive_prefix_tree(ts):
    n = len(ts)
    incl = list(ts)
    d = 1
    while d < n:
        incl = [incl[j] if j < d else incl[j] + incl[j - d]
                for j in range(n)]
        d *= 2
    return [None] + incl[:-1]


def _sc_row_cumsum(x):
    mesh = plsc.VectorSubcoreMesh(
        core_axis_name="c", subcore_axis_name="s")

    @functools.partial(
        pl.kernel,
        out_type=jax.ShapeDtypeStruct((ROWS, COLS), jnp.float32),
        mesh=mesh,
        scratch_types=[
            pltpu.VMEM((2, COLS), jnp.float32),
            pltpu.SemaphoreType.DMA,
            pltpu.SemaphoreType.DMA,
            pltpu.SemaphoreType.DMA,
            pltpu.SemaphoreType.DMA,
        ],
        compiler_params=pltpu.CompilerParams(needs_layout_passes=False),
    )
    def k(x_hbm, out_hbm, buf, in_sem0, in_sem1, out_sem0, out_sem1):
        wid = lax.axis_index("s") * NUM_CORES + lax.axis_index("c")
        idx_last = jnp.full((LANES,), LANES - 1, jnp.int32)
        in_sems = (in_sem0, in_sem1)
        out_sems = (out_sem0, out_sem1)

        def row_idx(r):
            return wid * ROWS_PER_WORKER + r

        def scan_row(b):
            def group_body(g, c):
                base = g * (UNROLL * LANES)
                sls = [pl.ds(base + j * LANES, LANES) for j in range(UNROLL)]
                ss = [plsc.cumsum(buf[b, sl]) for sl in sls]
                ts = [s.at[idx_last].get(mode="promise_in_bounds")
                      for s in ss]
                e = _exclusive_prefix_tree(ts)
                pres = [c] + [c + e[j] for j in range(1, UNROLL)]
                for j in range(UNROLL):
                    buf[b, sls[j]] = ss[j] + pres[j]
                return c + (e[UNROLL - 1] + ts[UNROLL - 1])

            plsc.parallel_loop(
                0, NGROUPS, 1, carry=jnp.zeros((LANES,), jnp.float32)
            )(group_body)

        pending_out = [None, None]
        copy_in = pltpu.async_copy(
            x_hbm.at[row_idx(0)], buf.at[0], in_sems[0])
        for r in range(ROWS_PER_WORKER):
            b = r % 2
            nb = (r + 1) % 2
            if r + 1 < ROWS_PER_WORKER:
                if pending_out[nb] is not None:
                    pending_out[nb].wait()
                    pending_out[nb] = None
                next_in = pltpu.async_copy(
                    x_hbm.at[row_idx(r + 1)], buf.at[nb], in_sems[nb])
            copy_in.wait()
            scan_row(b)
            pending_out[b] = pltpu.async_copy(
                buf.at[b], out_hbm.at[row_idx(r)], out_sems[b])
            if r + 1 < ROWS_PER_WORKER:
                copy_in = next_in
        for p in pending_out:
            if p is not None:
                p.wait()

    return k(x)


def kernel(x):
    return _sc_row_cumsum(x)
